# baseline (speedup 1.0000x reference)
import jax
import jax.numpy as jnp
from jax import lax
from jax.experimental import pallas as pl
from jax.experimental.pallas import tpu as pltpu

N = 4096
NU = 16
ID = 2
B = 2
F = (ID + NU) * B
BLK = 256
NBLK = N // BLK
CH = 512


def _dcgru_body(adj_ref, inp_ref, hx_ref, wr_ref, br_ref, wc_ref, bc_ref,
                out_ref, bmat_ref, x0c_ref, x0f_ref, acc1_ref, xb_ref):
    i = pl.program_id(0)

    @pl.when(i == 0)
    def _init():
        hxv = hx_ref[...]
        inv = inp_ref[...]
        t0 = lax.transpose(hxv[0:N, :], (1, 0))
        t1 = lax.transpose(hxv[N:2 * N, :], (1, 0))
        it = lax.transpose(inv, (1, 0)).reshape(ID * B, N)
        xv = jnp.concatenate([t0, t1, it], axis=0)
        x0f_ref[...] = xv
        xvb = xv.astype(jnp.bfloat16)
        for k in range(NBLK):
            x0c_ref[k] = xvb[:, k * BLK:(k + 1) * BLK]
        acc1_ref[...] = jnp.zeros((F, N), jnp.float32)

    blk = adj_ref[...]
    s = jnp.sum(blk, axis=1, keepdims=True)
    dinv = 1.0 / (s + 1.0)
    rows = lax.broadcasted_iota(jnp.int32, (BLK, N), 0) + i * BLK
    cols = lax.broadcasted_iota(jnp.int32, (BLK, N), 1)
    eye = (rows == cols).astype(jnp.float32)
    scaled = ((blk + eye) * dinv).astype(jnp.bfloat16)
    bmat_ref[pl.ds(i * BLK, BLK), :] = scaled
    acc1_ref[...] += lax.dot_general(x0c_ref[i], scaled,
                                     (((1,), (0,)), ((), ())),
                                     preferred_element_type=jnp.float32)

    @pl.when(i == NBLK - 1)
    def _compute():
        x0a = x0f_ref[...]

        def matmul_b(x):
            xb = x.astype(jnp.bfloat16)
            for k in range(N // CH):
                xb_ref[k] = xb[:, k * CH:(k + 1) * CH]

            def step(k, acc):
                bs = bmat_ref[pl.ds(k * CH, CH), :]
                return acc + lax.dot_general(xb_ref[k], bs,
                                             (((1,), (0,)), ((), ())),
                                             preferred_element_type=jnp.float32)

            return lax.fori_loop(0, N // CH, step,
                                 jnp.zeros((F, N), jnp.float32))

        def dense(w_ref, b_ref, x0, x1, x2):
            wv = w_ref[...]
            acc = lax.dot_general(wv[:, 0:F], x0, (((1,), (0,)), ((), ())),
                                  preferred_element_type=jnp.float32)
            acc += lax.dot_general(wv[:, F:2 * F], x1, (((1,), (0,)), ((), ())),
                                   preferred_element_type=jnp.float32)
            acc += lax.dot_general(wv[:, 2 * F:3 * F], x2,
                                   (((1,), (0,)), ((), ())),
                                   preferred_element_type=jnp.float32)
            return acc + b_ref[...]

        x1a = acc1_ref[...]
        x2a = 2.0 * matmul_b(x1a) - x0a
        val = jax.nn.sigmoid(dense(wr_ref, br_ref, x0a, x1a, x2a))
        r = jnp.concatenate([val[0:NU, :], val[2 * NU:3 * NU, :]], axis=0)
        u = jnp.concatenate([val[NU:2 * NU, :], val[3 * NU:4 * NU, :]], axis=0)

        hx = x0a[0:NU * B, :]
        x0b = jnp.concatenate([r * hx, x0a[NU * B:F, :]], axis=0)
        x1b = matmul_b(x0b)
        x2b = 2.0 * matmul_b(x1b) - x0b
        c = jnp.tanh(dense(wc_ref, bc_ref, x0b, x1b, x2b))

        h = u * hx + (1.0 - u) * c
        out_ref[0] = lax.transpose(h[0:NU, :], (1, 0))
        out_ref[1] = lax.transpose(h[NU:2 * NU, :], (1, 0))


def _prep_weights(W, bias, out_units):
    Wr = W.reshape(ID + NU, 3, out_units)
    eye = jnp.eye(B, dtype=W.dtype)
    state = jnp.einsum('umo,bd->bomdu', Wr[ID:], eye)
    state = state.reshape(B * out_units, 3, B * NU)
    inp = jnp.einsum('cmo,bd->bomcd', Wr[:ID], eye)
    inp = inp.reshape(B * out_units, 3, B * ID)
    wcat = jnp.concatenate([state, inp], axis=2).reshape(B * out_units, 3 * F)
    brow = jnp.tile(bias, B).reshape(B * out_units, 1)
    return wcat, brow


@jax.jit
def kernel(inputs, hx, adj, W_ru, b_ru, W_c, b_c):
    inp2 = inputs.reshape(B * N, ID)
    hx2 = hx.reshape(B * N, NU)
    wr, brow_r = _prep_weights(W_ru, b_ru, 2 * NU)
    wc, brow_c = _prep_weights(W_c, b_c, NU)

    full = lambda shape: pl.BlockSpec(shape, lambda i: tuple(0 for _ in shape))
    out = pl.pallas_call(
        _dcgru_body,
        grid=(NBLK,),
        in_specs=[
            pl.BlockSpec((BLK, N), lambda i: (i, 0)),
            full((B * N, ID)),
            full((B * N, NU)),
            full((4 * NU, 3 * F)), full((4 * NU, 1)),
            full((2 * NU, 3 * F)), full((2 * NU, 1)),
        ],
        out_specs=full((B, N, NU)),
        out_shape=jax.ShapeDtypeStruct((B, N, NU), jnp.float32),
        scratch_shapes=[
            pltpu.VMEM((N, N), jnp.bfloat16),
            pltpu.VMEM((NBLK, F, BLK), jnp.bfloat16),
            pltpu.VMEM((F, N), jnp.float32),
            pltpu.VMEM((F, N), jnp.float32),
            pltpu.VMEM((N // CH, F, CH), jnp.bfloat16),
        ],
        compiler_params=pltpu.CompilerParams(
            dimension_semantics=("arbitrary",),
            vmem_limit_bytes=128 * 1024 * 1024,
        ),
    )(adj, inp2, hx2, wr, brow_r, wc, brow_c)

    return out.reshape(B, N * NU)


# speedup vs baseline: 1.0468x; 1.0468x over previous
import jax
import jax.numpy as jnp
from jax import lax
from jax.experimental import pallas as pl
from jax.experimental.pallas import tpu as pltpu

N = 4096
NU = 16
ID = 2
B = 2
F = (ID + NU) * B
BLK = 256
NBLK = N // BLK
CH = 1024


def _dcgru_body(adj_ref, inp_ref, hx_ref, wr_ref, br_ref, wc_ref, bc_ref,
                out_ref, bmat_ref, x0c_ref, x0f_ref, acc1_ref, xb_ref):
    i = pl.program_id(0)

    @pl.when(i == 0)
    def _init():
        hxv = hx_ref[...]
        inv = inp_ref[...]
        t0 = lax.transpose(hxv[0:N, :], (1, 0))
        t1 = lax.transpose(hxv[N:2 * N, :], (1, 0))
        it = lax.transpose(inv, (1, 0)).reshape(ID * B, N)
        xv = jnp.concatenate([t0, t1, it], axis=0)
        x0f_ref[...] = xv
        xvb = xv.astype(jnp.bfloat16)
        for k in range(NBLK):
            x0c_ref[k] = xvb[:, k * BLK:(k + 1) * BLK]
        acc1_ref[...] = jnp.zeros((F, N), jnp.float32)

    blk = adj_ref[...]
    s = jnp.sum(blk, axis=1, keepdims=True)
    dinv = 1.0 / (s + 1.0)
    rows = lax.broadcasted_iota(jnp.int32, (BLK, N), 0) + i * BLK
    cols = lax.broadcasted_iota(jnp.int32, (BLK, N), 1)
    eye = (rows == cols).astype(jnp.float32)
    scaled = ((blk + eye) * dinv).astype(jnp.bfloat16)
    bmat_ref[pl.ds(i * BLK, BLK), :] = scaled
    acc1_ref[...] += lax.dot_general(x0c_ref[i], scaled,
                                     (((1,), (0,)), ((), ())),
                                     preferred_element_type=jnp.float32)

    @pl.when(i == NBLK - 1)
    def _compute():
        x0a = x0f_ref[...]

        def matmul_b(x):
            xb = x.astype(jnp.bfloat16)
            for k in range(N // CH):
                xb_ref[k] = xb[:, k * CH:(k + 1) * CH]

            def step(k, acc):
                bs = bmat_ref[pl.ds(k * CH, CH), :]
                return acc + lax.dot_general(xb_ref[k], bs,
                                             (((1,), (0,)), ((), ())),
                                             preferred_element_type=jnp.float32)

            return lax.fori_loop(0, N // CH, step,
                                 jnp.zeros((F, N), jnp.float32), unroll=2)

        def dense(w_ref, b_ref, x0, x1, x2):
            wv = w_ref[...]
            acc = lax.dot_general(wv[:, 0:F], x0, (((1,), (0,)), ((), ())),
                                  preferred_element_type=jnp.float32)
            acc += lax.dot_general(wv[:, F:2 * F], x1, (((1,), (0,)), ((), ())),
                                   preferred_element_type=jnp.float32)
            acc += lax.dot_general(wv[:, 2 * F:3 * F], x2,
                                   (((1,), (0,)), ((), ())),
                                   preferred_element_type=jnp.float32)
            return acc + b_ref[...]

        x1a = acc1_ref[...]
        x2a = 2.0 * matmul_b(x1a) - x0a
        val = jax.nn.sigmoid(dense(wr_ref, br_ref, x0a, x1a, x2a))
        r = jnp.concatenate([val[0:NU, :], val[2 * NU:3 * NU, :]], axis=0)
        u = jnp.concatenate([val[NU:2 * NU, :], val[3 * NU:4 * NU, :]], axis=0)

        hx = x0a[0:NU * B, :]
        x0b = jnp.concatenate([r * hx, x0a[NU * B:F, :]], axis=0)
        x1b = matmul_b(x0b)
        x2b = 2.0 * matmul_b(x1b) - x0b
        c = jnp.tanh(dense(wc_ref, bc_ref, x0b, x1b, x2b))

        h = u * hx + (1.0 - u) * c
        out_ref[0] = lax.transpose(h[0:NU, :], (1, 0))
        out_ref[1] = lax.transpose(h[NU:2 * NU, :], (1, 0))


def _prep_weights(W, bias, out_units):
    Wr = W.reshape(ID + NU, 3, out_units)
    eye = jnp.eye(B, dtype=W.dtype)
    state = jnp.einsum('umo,bd->bomdu', Wr[ID:], eye)
    state = state.reshape(B * out_units, 3, B * NU)
    inp = jnp.einsum('cmo,bd->bomcd', Wr[:ID], eye)
    inp = inp.reshape(B * out_units, 3, B * ID)
    wcat = jnp.concatenate([state, inp], axis=2).reshape(B * out_units, 3 * F)
    brow = jnp.tile(bias, B).reshape(B * out_units, 1)
    return wcat, brow


@jax.jit
def kernel(inputs, hx, adj, W_ru, b_ru, W_c, b_c):
    inp2 = inputs.reshape(B * N, ID)
    hx2 = hx.reshape(B * N, NU)
    wr, brow_r = _prep_weights(W_ru, b_ru, 2 * NU)
    wc, brow_c = _prep_weights(W_c, b_c, NU)

    full = lambda shape: pl.BlockSpec(shape, lambda i: tuple(0 for _ in shape))
    out = pl.pallas_call(
        _dcgru_body,
        grid=(NBLK,),
        in_specs=[
            pl.BlockSpec((BLK, N), lambda i: (i, 0)),
            full((B * N, ID)),
            full((B * N, NU)),
            full((4 * NU, 3 * F)), full((4 * NU, 1)),
            full((2 * NU, 3 * F)), full((2 * NU, 1)),
        ],
        out_specs=full((B, N, NU)),
        out_shape=jax.ShapeDtypeStruct((B, N, NU), jnp.float32),
        scratch_shapes=[
            pltpu.VMEM((N, N), jnp.bfloat16),
            pltpu.VMEM((NBLK, F, BLK), jnp.bfloat16),
            pltpu.VMEM((F, N), jnp.float32),
            pltpu.VMEM((F, N), jnp.float32),
            pltpu.VMEM((N // CH, F, CH), jnp.bfloat16),
        ],
        compiler_params=pltpu.CompilerParams(
            dimension_semantics=("arbitrary",),
            vmem_limit_bytes=128 * 1024 * 1024,
        ),
    )(adj, inp2, hx2, wr, brow_r, wc, brow_c)

    return out.reshape(B, N * NU)
